# Initial kernel scaffold; baseline (speedup 1.0000x reference)
import jax
import jax.numpy as jnp
from jax.experimental import pallas as pl


def kernel(x, edge_attr, lin0_W, lin0_b, enn_W1, enn_b1, enn_W2, enn_b2, root_W, conv_b, lin1_W, lin1_b, lin2_W, lin2_b, edge_index):
    raise NotImplementedError("write your pallas kernel here")


# speedup vs baseline: 1.7947x; 1.7947x over previous
import jax
import jax.numpy as jnp
from jax import lax
from jax.experimental import pallas as pl
from jax.experimental.pallas import tpu as pltpu
from jax.experimental.pallas import tpu_sc as plsc

N = 10000
E = 160000
D = 128
H = 16
NC = 2
NS = 16
NW = NC * NS
EW = E // NW
CW = 40
CH = EW // CW
ROWS_N = N // NS
TE = 2000
NT = 10
NB = N // NT

_mesh = plsc.VectorSubcoreMesh(core_axis_name="c", subcore_axis_name="s",
                               num_cores=NC, num_subcores=NS)


def _make_gather(n_pass, chp):
    ch = n_pass * chp

    def body(table_hbm, idx_hbm, out_hbm, idx_v, rows_v, sem):
        wid = lax.axis_index("s") * NC + lax.axis_index("c")
        pltpu.sync_copy(idx_hbm.at[wid], idx_v)
        for p in range(n_pass):
            def fire(j, carry):
                pltpu.async_copy(table_hbm.at[idx_v.at[p * chp + j]],
                                 rows_v.at[j], sem)
                return carry
            lax.fori_loop(0, chp, fire, 0)
            pltpu.make_async_copy(out_hbm.at[wid, pl.ds(p * chp, chp)],
                                  rows_v, sem).wait()
            pltpu.sync_copy(rows_v, out_hbm.at[wid, pl.ds(p * chp, chp)])

    def run(table, idx3):
        return pl.kernel(
            body,
            out_type=jax.ShapeDtypeStruct((NW, ch, CW, H), jnp.float32),
            mesh=_mesh,
            scratch_types=[pltpu.VMEM((ch, CW), jnp.int32),
                           pltpu.VMEM((chp, CW, H), jnp.float32),
                           pltpu.SemaphoreType.DMA],
            compiler_params=pltpu.CompilerParams(use_tc_tiling_on_sc=False),
        )(table, idx3)
    return run


_gather_e = _make_gather(1, CH)
_gather_2e = _make_gather(2, CH)


def _scatter_add(msg4, dst3, zeros_nh):
    def body(msg_hbm, dst_hbm, z_hbm, out_hbm, idx_v, msg_v, agg_sh):
        c = lax.axis_index("c")
        s = lax.axis_index("s")
        wid = s * NC + c
        pltpu.sync_copy(z_hbm.at[pl.ds(s * ROWS_N, ROWS_N)],
                        agg_sh.at[pl.ds(s * ROWS_N, ROWS_N)])
        pltpu.sync_copy(dst_hbm.at[wid], idx_v)
        pltpu.sync_copy(msg_hbm.at[wid], msg_v)
        plsc.subcore_barrier()

        def sc(j, carry):
            pltpu.sync_copy(msg_v.at[j], agg_sh.at[idx_v.at[j]], add=True)
            return carry
        lax.fori_loop(0, CH, sc, 0)
        plsc.subcore_barrier()
        pltpu.sync_copy(agg_sh.at[pl.ds(s * ROWS_N, ROWS_N)],
                        out_hbm.at[c, pl.ds(s * ROWS_N, ROWS_N)])

    return pl.kernel(
        body,
        out_type=jax.ShapeDtypeStruct((NC, N, H), jnp.float32),
        mesh=_mesh,
        scratch_types=[pltpu.VMEM((CH, CW), jnp.int32),
                       pltpu.VMEM((CH, CW, H), jnp.float32),
                       pltpu.VMEM_SHARED((N, H), jnp.float32)],
        compiler_params=pltpu.CompilerParams(use_tc_tiling_on_sc=False),
    )(msg4, dst3, zeros_nh)


def _tc_nodes(x, w0, b0, wr, br):
    def body(x_ref, w0_ref, b0_ref, wr_ref, br_ref, h_ref, hr_ref):
        h = jnp.maximum(
            jnp.dot(x_ref[...], w0_ref[...],
                    preferred_element_type=jnp.float32) + b0_ref[...], 0.0)
        h_ref[...] = h
        hr_ref[...] = jnp.dot(h, wr_ref[...],
                              preferred_element_type=jnp.float32) + br_ref[...]
    return pl.pallas_call(
        body,
        grid=(NT,),
        in_specs=[pl.BlockSpec((NB, D), lambda i: (i, 0)),
                  pl.BlockSpec((D, H), lambda i: (0, 0)),
                  pl.BlockSpec((1, H), lambda i: (0, 0)),
                  pl.BlockSpec((H, H), lambda i: (0, 0)),
                  pl.BlockSpec((1, H), lambda i: (0, 0))],
        out_specs=[pl.BlockSpec((NB, H), lambda i: (i, 0)),
                   pl.BlockSpec((NB, H), lambda i: (i, 0))],
        out_shape=[jax.ShapeDtypeStruct((N, H), jnp.float32),
                   jax.ShapeDtypeStruct((N, H), jnp.float32)],
    )(x, w0, b0, wr, br)


def _tc_msg(ea, xj, w1, b1, w2p, b2r):
    def body(ea_ref, xj_ref, w1_ref, b1_ref, w2p_ref, b2r_ref, msg_ref):
        z = jnp.maximum(
            jnp.dot(ea_ref[...], w1_ref[...],
                    preferred_element_type=jnp.float32) + b1_ref[...], 0.0)
        xj = xj_ref[...]
        p = (z[:, :, None] * xj[:, None, :]).reshape(TE, H * H)
        msg_ref[...] = (
            jnp.dot(p, w2p_ref[...], preferred_element_type=jnp.float32)
            + jnp.dot(xj, b2r_ref[...], preferred_element_type=jnp.float32))
    return pl.pallas_call(
        body,
        grid=(E // TE,),
        in_specs=[pl.BlockSpec((TE, H), lambda i: (i, 0)),
                  pl.BlockSpec((TE, H), lambda i: (i, 0)),
                  pl.BlockSpec((H, H), lambda i: (0, 0)),
                  pl.BlockSpec((1, H), lambda i: (0, 0)),
                  pl.BlockSpec((H * H, H), lambda i: (0, 0)),
                  pl.BlockSpec((H, H), lambda i: (0, 0))],
        out_specs=pl.BlockSpec((TE, H), lambda i: (i, 0)),
        out_shape=jax.ShapeDtypeStruct((E, H), jnp.float32),
    )(ea, xj, w1, b1, w2p, b2r)


def _tc_combine(agg2, hr):
    def body(a_ref, hr_ref, o_ref):
        o_ref[...] = a_ref[0] + a_ref[1] + hr_ref[...]
    return pl.pallas_call(
        body,
        grid=(NT,),
        in_specs=[pl.BlockSpec((NC, NB, H), lambda i: (0, i, 0)),
                  pl.BlockSpec((NB, H), lambda i: (i, 0))],
        out_specs=pl.BlockSpec((NB, H), lambda i: (i, 0)),
        out_shape=jax.ShapeDtypeStruct((N, H), jnp.float32),
    )(agg2, hr)


def _tc_head(r2, w1, b1, w2t, b2):
    def body(xs_ref, xd_ref, w1_ref, b1_ref, w2t_ref, b2_ref, o_ref):
        emb = xs_ref[0] * xd_ref[0]
        e1 = jnp.maximum(
            jnp.dot(emb, w1_ref[...],
                    preferred_element_type=jnp.float32) + b1_ref[...], 0.0)
        o_ref[...] = (jnp.sum(e1 * w2t_ref[...], axis=1, keepdims=True)
                      + b2_ref[...])
    return pl.pallas_call(
        body,
        grid=(E // TE,),
        in_specs=[pl.BlockSpec((1, TE, H), lambda i: (0, i, 0)),
                  pl.BlockSpec((1, TE, H), lambda i: (1, i, 0)),
                  pl.BlockSpec((H, 8), lambda i: (0, 0)),
                  pl.BlockSpec((1, 8), lambda i: (0, 0)),
                  pl.BlockSpec((1, 8), lambda i: (0, 0)),
                  pl.BlockSpec((1, 1), lambda i: (0, 0))],
        out_specs=pl.BlockSpec((TE, 1), lambda i: (i, 0)),
        out_shape=jax.ShapeDtypeStruct((E, 1), jnp.float32),
    )(r2, r2, w1, b1, w2t, b2)


def kernel(x, edge_attr, lin0_W, lin0_b, enn_W1, enn_b1, enn_W2, enn_b2,
           root_W, conv_b, lin1_W, lin1_b, lin2_W, lin2_b, edge_index):
    src3 = edge_index[0].reshape(NW, CH, CW)
    dst3 = edge_index[1].reshape(NW, CH, CW)
    ei3 = edge_index.reshape(NW, 2 * CH, CW)

    h, hr = _tc_nodes(x, lin0_W, lin0_b.reshape(1, H),
                      root_W, conv_b.reshape(1, H))
    xj = _gather_e(h, src3).reshape(E, H)
    msg = _tc_msg(edge_attr, xj, enn_W1, enn_b1.reshape(1, H),
                  enn_W2.reshape(H * H, H), enn_b2.reshape(H, H))
    agg2 = _scatter_add(msg.reshape(NW, CH, CW, H), dst3,
                        jnp.zeros((N, H), jnp.float32))
    out = _tc_combine(agg2, hr)
    r2 = _gather_2e(out, ei3).reshape(2, E, H)
    score = _tc_head(r2, lin1_W, lin1_b.reshape(1, 8),
                     lin2_W.reshape(1, 8), lin2_b.reshape(1, 1))
    return score.reshape(E)


# speedup vs baseline: 3.1713x; 1.7670x over previous
import numpy as np
import jax
import jax.numpy as jnp
from jax import lax
from jax.experimental import pallas as pl
from jax.experimental.pallas import tpu as pltpu
from jax.experimental.pallas import tpu_sc as plsc

N = 10000
E = 160000
D = 128
H = 16
NC = 2
NS = 16
NW = NC * NS
EW = E // NW
CW = 40
CH = EW // CW
ROWS_N = N // NS
TE = 2000
NT = 10
NB = N // NT

_mesh = plsc.VectorSubcoreMesh(core_axis_name="c", subcore_axis_name="s",
                               num_cores=NC, num_subcores=NS)


def _make_gather(n_pass, chp):
    ch = n_pass * chp

    def body(table_hbm, idx_hbm, out_hbm, idx_v, rows_v, sem):
        wid = lax.axis_index("s") * NC + lax.axis_index("c")
        pltpu.sync_copy(idx_hbm.at[wid], idx_v)
        for p in range(n_pass):
            def fire(j, carry):
                pltpu.async_copy(table_hbm.at[idx_v.at[p * chp + j]],
                                 rows_v.at[j], sem)
                return carry
            lax.fori_loop(0, chp, fire, 0)
            pltpu.make_async_copy(out_hbm.at[wid, pl.ds(p * chp, chp)],
                                  rows_v, sem).wait()
            pltpu.sync_copy(rows_v, out_hbm.at[wid, pl.ds(p * chp, chp)])

    def run(table, idx3):
        return pl.kernel(
            body,
            out_type=jax.ShapeDtypeStruct((NW, ch, CW, H), jnp.float32),
            mesh=_mesh,
            scratch_types=[pltpu.VMEM((ch, CW), jnp.int32),
                           pltpu.VMEM((chp, CW, H), jnp.float32),
                           pltpu.SemaphoreType.DMA],
            compiler_params=pltpu.CompilerParams(use_tc_tiling_on_sc=False),
        )(table, idx3)
    return run


_gather_e = _make_gather(1, CH)
_gather_2e = _make_gather(2, CH)


def _scatter_add(msg4, dst3, zeros_nh):
    def body(msg_hbm, dst_hbm, z_hbm, out_hbm, idx_v, msg_v, agg_sh):
        c = lax.axis_index("c")
        s = lax.axis_index("s")
        wid = s * NC + c
        pltpu.sync_copy(z_hbm.at[pl.ds(s * ROWS_N, ROWS_N)],
                        agg_sh.at[pl.ds(s * ROWS_N, ROWS_N)])
        pltpu.sync_copy(dst_hbm.at[wid], idx_v)
        pltpu.sync_copy(msg_hbm.at[wid], msg_v)
        plsc.subcore_barrier()

        def sc(j, carry):
            pltpu.sync_copy(msg_v.at[j], agg_sh.at[idx_v.at[j]], add=True)
            return carry
        lax.fori_loop(0, CH, sc, 0)
        plsc.subcore_barrier()
        pltpu.sync_copy(agg_sh.at[pl.ds(s * ROWS_N, ROWS_N)],
                        out_hbm.at[c, pl.ds(s * ROWS_N, ROWS_N)])

    return pl.kernel(
        body,
        out_type=jax.ShapeDtypeStruct((NC, N, H), jnp.float32),
        mesh=_mesh,
        scratch_types=[pltpu.VMEM((CH, CW), jnp.int32),
                       pltpu.VMEM((CH, CW, H), jnp.float32),
                       pltpu.VMEM_SHARED((N, H), jnp.float32)],
        compiler_params=pltpu.CompilerParams(use_tc_tiling_on_sc=False),
    )(msg4, dst3, zeros_nh)


def _tc_nodes(x, w0, b0, wr, br):
    def body(x_ref, w0_ref, b0_ref, wr_ref, br_ref, h_ref, hr_ref):
        h = jnp.maximum(
            jnp.dot(x_ref[...], w0_ref[...],
                    preferred_element_type=jnp.float32) + b0_ref[...], 0.0)
        h_ref[...] = h
        hr_ref[...] = jnp.dot(h, wr_ref[...],
                              preferred_element_type=jnp.float32) + br_ref[...]
    return pl.pallas_call(
        body,
        grid=(NT,),
        in_specs=[pl.BlockSpec((NB, D), lambda i: (i, 0)),
                  pl.BlockSpec((D, H), lambda i: (0, 0)),
                  pl.BlockSpec((1, H), lambda i: (0, 0)),
                  pl.BlockSpec((H, H), lambda i: (0, 0)),
                  pl.BlockSpec((1, H), lambda i: (0, 0))],
        out_specs=[pl.BlockSpec((NB, H), lambda i: (i, 0)),
                   pl.BlockSpec((NB, H), lambda i: (i, 0))],
        out_shape=[jax.ShapeDtypeStruct((N, H), jnp.float32),
                   jax.ShapeDtypeStruct((N, H), jnp.float32)],
    )(x, w0, b0, wr, br)


def _tc_msg(ea, xj, w1, b1, w2p, b2r):
    rk = jnp.asarray(np.repeat(np.eye(H, dtype=np.float32), H, axis=1))
    ri = jnp.asarray(np.tile(np.eye(H, dtype=np.float32), (1, H)))

    def body(ea_ref, xj_ref, w1_ref, b1_ref, w2p_ref, b2r_ref, rk_ref,
             ri_ref, msg_ref):
        z = jnp.maximum(
            jnp.dot(ea_ref[...], w1_ref[...],
                    preferred_element_type=jnp.float32) + b1_ref[...], 0.0)
        xj = xj_ref[...]
        p = (jnp.dot(z, rk_ref[...], preferred_element_type=jnp.float32)
             * jnp.dot(xj, ri_ref[...], preferred_element_type=jnp.float32))
        msg_ref[...] = (
            jnp.dot(p, w2p_ref[...], preferred_element_type=jnp.float32)
            + jnp.dot(xj, b2r_ref[...], preferred_element_type=jnp.float32))
    return pl.pallas_call(
        body,
        grid=(E // TE,),
        in_specs=[pl.BlockSpec((TE, H), lambda i: (i, 0)),
                  pl.BlockSpec((TE, H), lambda i: (i, 0)),
                  pl.BlockSpec((H, H), lambda i: (0, 0)),
                  pl.BlockSpec((1, H), lambda i: (0, 0)),
                  pl.BlockSpec((H * H, H), lambda i: (0, 0)),
                  pl.BlockSpec((H, H), lambda i: (0, 0)),
                  pl.BlockSpec((H, H * H), lambda i: (0, 0)),
                  pl.BlockSpec((H, H * H), lambda i: (0, 0))],
        out_specs=pl.BlockSpec((TE, H), lambda i: (i, 0)),
        out_shape=jax.ShapeDtypeStruct((E, H), jnp.float32),
    )(ea, xj, w1, b1, w2p, b2r, rk, ri)


def _tc_combine(agg2, hr):
    def body(a_ref, hr_ref, o_ref):
        o_ref[...] = a_ref[0] + a_ref[1] + hr_ref[...]
    return pl.pallas_call(
        body,
        grid=(NT,),
        in_specs=[pl.BlockSpec((NC, NB, H), lambda i: (0, i, 0)),
                  pl.BlockSpec((NB, H), lambda i: (i, 0))],
        out_specs=pl.BlockSpec((NB, H), lambda i: (i, 0)),
        out_shape=jax.ShapeDtypeStruct((N, H), jnp.float32),
    )(agg2, hr)


def _tc_head(r2, w1, b1, w2t, b2):
    def body(xs_ref, xd_ref, w1_ref, b1_ref, w2t_ref, b2_ref, o_ref):
        emb = xs_ref[0] * xd_ref[0]
        e1 = jnp.maximum(
            jnp.dot(emb, w1_ref[...],
                    preferred_element_type=jnp.float32) + b1_ref[...], 0.0)
        o_ref[...] = (jnp.sum(e1 * w2t_ref[...], axis=1, keepdims=True)
                      + b2_ref[...])
    return pl.pallas_call(
        body,
        grid=(E // TE,),
        in_specs=[pl.BlockSpec((1, TE, H), lambda i: (0, i, 0)),
                  pl.BlockSpec((1, TE, H), lambda i: (1, i, 0)),
                  pl.BlockSpec((H, 8), lambda i: (0, 0)),
                  pl.BlockSpec((1, 8), lambda i: (0, 0)),
                  pl.BlockSpec((1, 8), lambda i: (0, 0)),
                  pl.BlockSpec((1, 1), lambda i: (0, 0))],
        out_specs=pl.BlockSpec((TE, 1), lambda i: (i, 0)),
        out_shape=jax.ShapeDtypeStruct((E, 1), jnp.float32),
    )(r2, r2, w1, b1, w2t, b2)


def kernel(x, edge_attr, lin0_W, lin0_b, enn_W1, enn_b1, enn_W2, enn_b2,
           root_W, conv_b, lin1_W, lin1_b, lin2_W, lin2_b, edge_index):
    src3 = edge_index[0].reshape(NW, CH, CW)
    dst3 = edge_index[1].reshape(NW, CH, CW)
    ei3 = edge_index.reshape(NW, 2 * CH, CW)

    h, hr = _tc_nodes(x, lin0_W, lin0_b.reshape(1, H),
                      root_W, conv_b.reshape(1, H))
    xj = _gather_e(h, src3).reshape(E, H)
    msg = _tc_msg(edge_attr, xj, enn_W1, enn_b1.reshape(1, H),
                  enn_W2.reshape(H * H, H), enn_b2.reshape(H, H))
    agg2 = _scatter_add(msg.reshape(NW, CH, CW, H), dst3,
                        jnp.zeros((N, H), jnp.float32))
    out = _tc_combine(agg2, hr)
    r2 = _gather_2e(out, ei3).reshape(2, E, H)
    score = _tc_head(r2, lin1_W, lin1_b.reshape(1, 8),
                     lin2_W.reshape(1, 8), lin2_b.reshape(1, 1))
    return score.reshape(E)


# speedup vs baseline: 3.4426x; 1.0856x over previous
import numpy as np
import jax
import jax.numpy as jnp
from jax import lax
from jax.experimental import pallas as pl
from jax.experimental.pallas import tpu as pltpu
from jax.experimental.pallas import tpu_sc as plsc

N = 10000
E = 160000
D = 128
H = 16
NC = 2
NS = 16
NW = NC * NS
EW = E // NW
CW = 40
CH = EW // CW
ROWS_N = N // NS
TE = 4000
NT = 10
NB = N // NT

_mesh = plsc.VectorSubcoreMesh(core_axis_name="c", subcore_axis_name="s",
                               num_cores=NC, num_subcores=NS)


def _make_gather(n_pass, chp):
    ch = n_pass * chp
    rows_w = ch * CW
    rows_p = chp * CW

    def body(table_hbm, idx_hbm, out_hbm, idx_v, rows_v, sem):
        wid = lax.axis_index("s") * NC + lax.axis_index("c")
        pltpu.sync_copy(idx_hbm.at[wid], idx_v)
        for p in range(n_pass):
            def fire(j, carry):
                pltpu.async_copy(table_hbm.at[idx_v.at[p * chp + j]],
                                 rows_v.at[pl.ds(j * CW, CW)], sem)
                return carry
            lax.fori_loop(0, chp, fire, 0)
            base = wid * rows_w + p * rows_p
            pltpu.make_async_copy(out_hbm.at[pl.ds(base, rows_p)],
                                  rows_v, sem).wait()
            pltpu.sync_copy(rows_v, out_hbm.at[pl.ds(base, rows_p)])

    def run(table, idx3):
        return pl.kernel(
            body,
            out_type=jax.ShapeDtypeStruct((NW * rows_w, H), jnp.float32),
            mesh=_mesh,
            scratch_types=[pltpu.VMEM((ch, CW), jnp.int32),
                           pltpu.VMEM((rows_p, H), jnp.float32),
                           pltpu.SemaphoreType.DMA],
            compiler_params=pltpu.CompilerParams(use_tc_tiling_on_sc=False),
        )(table, idx3)
    return run


_gather_e = _make_gather(1, CH)
_gather_2e = _make_gather(2, CH)


def _scatter_add(msg, dst3, zeros_nh):
    def body(msg_hbm, dst_hbm, z_hbm, out_hbm, idx_v, msg_v, agg_sh):
        c = lax.axis_index("c")
        s = lax.axis_index("s")
        wid = s * NC + c
        pltpu.sync_copy(z_hbm.at[pl.ds(s * ROWS_N, ROWS_N)],
                        agg_sh.at[pl.ds(s * ROWS_N, ROWS_N)])
        pltpu.sync_copy(dst_hbm.at[wid], idx_v)
        pltpu.sync_copy(msg_hbm.at[pl.ds(wid * EW, EW)], msg_v)
        plsc.subcore_barrier()

        def sc(j, carry):
            pltpu.sync_copy(msg_v.at[pl.ds(j * CW, CW)],
                            agg_sh.at[idx_v.at[j]], add=True)
            return carry
        lax.fori_loop(0, CH, sc, 0)
        plsc.subcore_barrier()
        pltpu.sync_copy(agg_sh.at[pl.ds(s * ROWS_N, ROWS_N)],
                        out_hbm.at[c, pl.ds(s * ROWS_N, ROWS_N)])

    return pl.kernel(
        body,
        out_type=jax.ShapeDtypeStruct((NC, N, H), jnp.float32),
        mesh=_mesh,
        scratch_types=[pltpu.VMEM((CH, CW), jnp.int32),
                       pltpu.VMEM((EW, H), jnp.float32),
                       pltpu.VMEM_SHARED((N, H), jnp.float32)],
        compiler_params=pltpu.CompilerParams(use_tc_tiling_on_sc=False),
    )(msg, dst3, zeros_nh)


def _tc_nodes(x, w0, b0, wr, br):
    def body(x_ref, w0_ref, b0_ref, wr_ref, br_ref, h_ref, hr_ref):
        h = jnp.maximum(
            jnp.dot(x_ref[...], w0_ref[...],
                    preferred_element_type=jnp.float32) + b0_ref[...], 0.0)
        h_ref[...] = h
        hr_ref[...] = jnp.dot(h, wr_ref[...],
                              preferred_element_type=jnp.float32) + br_ref[...]
    return pl.pallas_call(
        body,
        grid=(NT,),
        in_specs=[pl.BlockSpec((NB, D), lambda i: (i, 0)),
                  pl.BlockSpec((D, H), lambda i: (0, 0)),
                  pl.BlockSpec((1, H), lambda i: (0, 0)),
                  pl.BlockSpec((H, H), lambda i: (0, 0)),
                  pl.BlockSpec((1, H), lambda i: (0, 0))],
        out_specs=[pl.BlockSpec((NB, H), lambda i: (i, 0)),
                   pl.BlockSpec((NB, H), lambda i: (i, 0))],
        out_shape=[jax.ShapeDtypeStruct((N, H), jnp.float32),
                   jax.ShapeDtypeStruct((N, H), jnp.float32)],
    )(x, w0, b0, wr, br)


def _tc_msg(ea, xj, w1, b1, w2p, b2r):
    rk = jnp.asarray(np.repeat(np.eye(H, dtype=np.float32), H, axis=1))
    ri = jnp.asarray(np.tile(np.eye(H, dtype=np.float32), (1, H)))

    def body(ea_ref, xj_ref, w1_ref, b1_ref, w2p_ref, b2r_ref, rk_ref,
             ri_ref, msg_ref):
        z = jnp.maximum(
            jnp.dot(ea_ref[...], w1_ref[...],
                    preferred_element_type=jnp.float32) + b1_ref[...], 0.0)
        xj = xj_ref[...]
        p = (jnp.dot(z, rk_ref[...], preferred_element_type=jnp.float32)
             * jnp.dot(xj, ri_ref[...], preferred_element_type=jnp.float32))
        msg_ref[...] = (
            jnp.dot(p, w2p_ref[...], preferred_element_type=jnp.float32)
            + jnp.dot(xj, b2r_ref[...], preferred_element_type=jnp.float32))
    return pl.pallas_call(
        body,
        grid=(E // TE,),
        in_specs=[pl.BlockSpec((TE, H), lambda i: (i, 0)),
                  pl.BlockSpec((TE, H), lambda i: (i, 0)),
                  pl.BlockSpec((H, H), lambda i: (0, 0)),
                  pl.BlockSpec((1, H), lambda i: (0, 0)),
                  pl.BlockSpec((H * H, H), lambda i: (0, 0)),
                  pl.BlockSpec((H, H), lambda i: (0, 0)),
                  pl.BlockSpec((H, H * H), lambda i: (0, 0)),
                  pl.BlockSpec((H, H * H), lambda i: (0, 0))],
        out_specs=pl.BlockSpec((TE, H), lambda i: (i, 0)),
        out_shape=jax.ShapeDtypeStruct((E, H), jnp.float32),
    )(ea, xj, w1, b1, w2p, b2r, rk, ri)


def _tc_combine(agg2, hr):
    def body(a_ref, hr_ref, o_ref):
        o_ref[...] = a_ref[0] + a_ref[1] + hr_ref[...]
    return pl.pallas_call(
        body,
        grid=(NT,),
        in_specs=[pl.BlockSpec((NC, NB, H), lambda i: (0, i, 0)),
                  pl.BlockSpec((NB, H), lambda i: (i, 0))],
        out_specs=pl.BlockSpec((NB, H), lambda i: (i, 0)),
        out_shape=jax.ShapeDtypeStruct((N, H), jnp.float32),
    )(agg2, hr)


def _tc_head(r2, w1, b1, w2t, b2):
    def body(xs_ref, xd_ref, w1_ref, b1_ref, w2t_ref, b2_ref, o_ref):
        emb = xs_ref[...] * xd_ref[...]
        e1 = jnp.maximum(
            jnp.dot(emb, w1_ref[...],
                    preferred_element_type=jnp.float32) + b1_ref[...], 0.0)
        o_ref[...] = (jnp.sum(e1 * w2t_ref[...], axis=1, keepdims=True)
                      + b2_ref[...])
    nblk = E // TE
    return pl.pallas_call(
        body,
        grid=(nblk,),
        in_specs=[pl.BlockSpec((TE, H), lambda i: (i, 0)),
                  pl.BlockSpec((TE, H), lambda i: (i + nblk, 0)),
                  pl.BlockSpec((H, 8), lambda i: (0, 0)),
                  pl.BlockSpec((1, 8), lambda i: (0, 0)),
                  pl.BlockSpec((1, 8), lambda i: (0, 0)),
                  pl.BlockSpec((1, 1), lambda i: (0, 0))],
        out_specs=pl.BlockSpec((TE, 1), lambda i: (i, 0)),
        out_shape=jax.ShapeDtypeStruct((E, 1), jnp.float32),
    )(r2, r2, w1, b1, w2t, b2)


def kernel(x, edge_attr, lin0_W, lin0_b, enn_W1, enn_b1, enn_W2, enn_b2,
           root_W, conv_b, lin1_W, lin1_b, lin2_W, lin2_b, edge_index):
    src3 = edge_index[0].reshape(NW, CH, CW)
    dst3 = edge_index[1].reshape(NW, CH, CW)
    ei3 = edge_index.reshape(NW, 2 * CH, CW)

    h, hr = _tc_nodes(x, lin0_W, lin0_b.reshape(1, H),
                      root_W, conv_b.reshape(1, H))
    xj = _gather_e(h, src3)
    msg = _tc_msg(edge_attr, xj, enn_W1, enn_b1.reshape(1, H),
                  enn_W2.reshape(H * H, H), enn_b2.reshape(H, H))
    agg2 = _scatter_add(msg, dst3, jnp.zeros((N, H), jnp.float32))
    out = _tc_combine(agg2, hr)
    r2 = _gather_2e(out, ei3)
    score = _tc_head(r2, lin1_W, lin1_b.reshape(1, 8),
                     lin2_W.reshape(1, 8), lin2_b.reshape(1, 1))
    return score.reshape(E)


# speedup vs baseline: 6.7757x; 1.9682x over previous
import numpy as np
import jax
import jax.numpy as jnp
from jax import lax
from jax.experimental import pallas as pl
from jax.experimental.pallas import tpu as pltpu
from jax.experimental.pallas import tpu_sc as plsc

N = 10000
NP = 10240
E = 160000
D = 128
H = 16
NC = 2
NS = 16
NW = NC * NS
EW = E // NW
CW = 40
CH = EW // CW
ROWS_N = NP // NS
TE = 6400
NT = 10
NB = N // NT

_mesh = plsc.VectorSubcoreMesh(core_axis_name="c", subcore_axis_name="s",
                               num_cores=NC, num_subcores=NS)


def _make_gather(n_pass, chp):
    ch = n_pass * chp
    rows_w = ch * CW
    rows_p = chp * CW

    def body(table_hbm, idx_hbm, out_hbm, idx_v, rows_v, sem):
        wid = lax.axis_index("s") * NC + lax.axis_index("c")
        pltpu.sync_copy(idx_hbm.at[wid], idx_v)
        for p in range(n_pass):
            def fire(j, carry):
                pltpu.async_copy(table_hbm.at[idx_v.at[p * chp + j]],
                                 rows_v.at[pl.ds(j * CW, CW)], sem)
                return carry
            lax.fori_loop(0, chp, fire, 0)
            base = wid * rows_w + p * rows_p
            pltpu.make_async_copy(out_hbm.at[pl.ds(base, rows_p)],
                                  rows_v, sem).wait()
            pltpu.sync_copy(rows_v, out_hbm.at[pl.ds(base, rows_p)])

    def run(table, idx3):
        return pl.kernel(
            body,
            out_type=jax.ShapeDtypeStruct((NW * rows_w, H), jnp.float32),
            mesh=_mesh,
            scratch_types=[pltpu.VMEM((ch, CW), jnp.int32),
                           pltpu.VMEM((rows_p, H), jnp.float32),
                           pltpu.SemaphoreType.DMA],
            compiler_params=pltpu.CompilerParams(use_tc_tiling_on_sc=False),
        )(table, idx3)
    return run


_gather_e = _make_gather(1, CH)
_gather_2e = _make_gather(2, CH)


def _scatter_add(msg, dst3, zeros_nh):
    def body(msg_hbm, dst_hbm, z_hbm, out_hbm, idx_v, msg_v, agg_sh):
        c = lax.axis_index("c")
        s = lax.axis_index("s")
        wid = s * NC + c
        pltpu.sync_copy(z_hbm.at[pl.ds(s * ROWS_N, ROWS_N)],
                        agg_sh.at[pl.ds(s * ROWS_N, ROWS_N)])
        pltpu.sync_copy(dst_hbm.at[wid], idx_v)
        pltpu.sync_copy(msg_hbm.at[pl.ds(wid * EW, EW)], msg_v)
        plsc.subcore_barrier()

        def sc(j, carry):
            pltpu.sync_copy(msg_v.at[pl.ds(j * CW, CW)],
                            agg_sh.at[idx_v.at[j]], add=True)
            return carry
        lax.fori_loop(0, CH, sc, 0)
        plsc.subcore_barrier()
        pltpu.sync_copy(agg_sh.at[pl.ds(s * ROWS_N, ROWS_N)],
                        out_hbm.at[c, pl.ds(s * ROWS_N, ROWS_N)])

    return pl.kernel(
        body,
        out_type=jax.ShapeDtypeStruct((NC, NP, H), jnp.float32),
        mesh=_mesh,
        scratch_types=[pltpu.VMEM((CH, CW), jnp.int32),
                       pltpu.VMEM((EW, H), jnp.float32),
                       pltpu.VMEM_SHARED((NP, H), jnp.float32)],
        compiler_params=pltpu.CompilerParams(use_tc_tiling_on_sc=False),
    )(msg, dst3, zeros_nh)


def _tc_nodes(x_pk, w0blk, b0t, wrbd, brt):
    def body(x_ref, w0_ref, b0_ref, wr_ref, br_ref, h_ref, hr_ref):
        h = jnp.maximum(
            jnp.dot(x_ref[...], w0_ref[...],
                    preferred_element_type=jnp.float32) + b0_ref[...], 0.0)
        h_ref[...] = h
        hr_ref[...] = jnp.dot(h, wr_ref[...],
                              preferred_element_type=jnp.float32) + br_ref[...]
    return pl.pallas_call(
        body,
        out_shape=[jax.ShapeDtypeStruct((NP // 8, 128), jnp.float32),
                   jax.ShapeDtypeStruct((NP // 8, 128), jnp.float32)],
    )(x_pk, w0blk, b0t, wrbd, brt)


TEP = TE // 8


def _tc_msg(ea_pk, xj_pk, w1bd, b1t, w2p, b2bd):
    rk = jnp.asarray(np.repeat(np.eye(H, dtype=np.float32), H, axis=1))
    ri = jnp.asarray(np.tile(np.eye(H, dtype=np.float32), (1, H)))
    ems = np.zeros((8 * 128, H), dtype=np.float32)
    for m in range(8):
        ems[m * 128 + m * 16:m * 128 + (m + 1) * 16, :] = np.eye(H)
    ems = jnp.asarray(ems)
    i128 = jnp.asarray(np.eye(128, dtype=np.float32))

    def body(ea_ref, xj_ref, w1_ref, b1_ref, w2p_ref, b2_ref, rk_ref,
             ri_ref, ems_ref, i128_ref, msg_ref):
        zpk = jnp.maximum(
            jnp.dot(ea_ref[...], w1_ref[...],
                    preferred_element_type=jnp.float32) + b1_ref[...], 0.0)
        xjpk = xj_ref[...]
        z = jnp.concatenate(
            [jnp.dot(zpk, ems_ref[m * 128:(m + 1) * 128, :],
                     preferred_element_type=jnp.float32) for m in range(8)],
            axis=0)
        xj = jnp.concatenate(
            [jnp.dot(xjpk, ems_ref[m * 128:(m + 1) * 128, :],
                     preferred_element_type=jnp.float32) for m in range(8)],
            axis=0)
        p = (jnp.dot(z, rk_ref[...], preferred_element_type=jnp.float32)
             * jnp.dot(xj, ri_ref[...], preferred_element_type=jnp.float32))
        msg = jnp.dot(p, w2p_ref[...], preferred_element_type=jnp.float32)
        mpk = jnp.dot(xjpk, b2_ref[...], preferred_element_type=jnp.float32)
        for m in range(8):
            mpk = mpk + jnp.dot(msg[m * TEP:(m + 1) * TEP, :],
                                i128_ref[m * 16:(m + 1) * 16, :],
                                preferred_element_type=jnp.float32)
        msg_ref[...] = mpk
    return pl.pallas_call(
        body,
        grid=(E // TE,),
        in_specs=[pl.BlockSpec((TEP, 128), lambda i: (i, 0)),
                  pl.BlockSpec((TEP, 128), lambda i: (i, 0)),
                  pl.BlockSpec((128, 128), lambda i: (0, 0)),
                  pl.BlockSpec((1, 128), lambda i: (0, 0)),
                  pl.BlockSpec((H * H, H), lambda i: (0, 0)),
                  pl.BlockSpec((128, 128), lambda i: (0, 0)),
                  pl.BlockSpec((H, H * H), lambda i: (0, 0)),
                  pl.BlockSpec((H, H * H), lambda i: (0, 0)),
                  pl.BlockSpec((8 * 128, H), lambda i: (0, 0)),
                  pl.BlockSpec((128, 128), lambda i: (0, 0))],
        out_specs=pl.BlockSpec((TEP, 128), lambda i: (i, 0)),
        out_shape=jax.ShapeDtypeStruct((E // 8, 128), jnp.float32),
    )(ea_pk, xj_pk, w1bd, b1t, w2p, b2bd, rk, ri, ems, i128)


def _tc_combine(agg2_pk, hr_pk):
    def body(a_ref, hr_ref, o_ref):
        o_ref[...] = a_ref[0] + a_ref[1] + hr_ref[...]
    return pl.pallas_call(
        body,
        out_shape=jax.ShapeDtypeStruct((NP // 8, 128), jnp.float32),
    )(agg2_pk, hr_pk)


def _tc_head(r2_pk, w1bd, b1t, w2sel, b2):
    def body(xs_ref, xd_ref, w1_ref, b1_ref, w2_ref, b2_ref, o_ref):
        emb = xs_ref[...] * xd_ref[...]
        e1 = jnp.maximum(
            jnp.dot(emb, w1_ref[...],
                    preferred_element_type=jnp.float32) + b1_ref[...], 0.0)
        o_ref[...] = jnp.dot(e1, w2_ref[...],
                             preferred_element_type=jnp.float32) + b2_ref[...]
    nblk = E // TE
    return pl.pallas_call(
        body,
        grid=(nblk,),
        in_specs=[pl.BlockSpec((TEP, 128), lambda i: (i, 0)),
                  pl.BlockSpec((TEP, 128), lambda i: (i + nblk, 0)),
                  pl.BlockSpec((128, 64), lambda i: (0, 0)),
                  pl.BlockSpec((1, 64), lambda i: (0, 0)),
                  pl.BlockSpec((64, 8), lambda i: (0, 0)),
                  pl.BlockSpec((1, 8), lambda i: (0, 0))],
        out_specs=pl.BlockSpec((TEP, 8), lambda i: (i, 0)),
        out_shape=jax.ShapeDtypeStruct((E // 8, 8), jnp.float32),
    )(r2_pk, r2_pk, w1bd, b1t, w2sel, b2)


def kernel(x, edge_attr, lin0_W, lin0_b, enn_W1, enn_b1, enn_W2, enn_b2,
           root_W, conv_b, lin1_W, lin1_b, lin2_W, lin2_b, edge_index):
    src3 = edge_index[0].reshape(NW, CH, CW)
    dst3 = edge_index[1].reshape(NW, CH, CW)
    ei3 = edge_index.reshape(NW, 2 * CH, CW)
    i8 = jnp.eye(8, dtype=jnp.float32)

    x_pk = jnp.pad(x, ((0, NP - N), (0, 0))).reshape(NP // 8, 8 * D)
    h_pk, hr_pk = _tc_nodes(
        x_pk, jnp.kron(i8, lin0_W), jnp.tile(lin0_b, 8).reshape(1, 128),
        jnp.kron(i8, root_W), jnp.tile(conv_b, 8).reshape(1, 128))
    xj = _gather_e(h_pk.reshape(NP, H), src3)
    msg_pk = _tc_msg(edge_attr.reshape(E // 8, 128), xj.reshape(E // 8, 128),
                     jnp.kron(i8, enn_W1), jnp.tile(enn_b1, 8).reshape(1, 128),
                     enn_W2.reshape(H * H, H),
                     jnp.kron(i8, enn_b2.reshape(H, H)))
    agg2 = _scatter_add(msg_pk.reshape(E, H), dst3,
                        jnp.zeros((NP, H), jnp.float32))
    out_pk = _tc_combine(agg2.reshape(NC, NP // 8, 128), hr_pk)
    r2 = _gather_2e(out_pk.reshape(NP, H), ei3)
    s8 = _tc_head(r2.reshape(2 * E // 8, 128),
                  jnp.kron(i8, lin1_W), jnp.tile(lin1_b, 8).reshape(1, 64),
                  jnp.kron(i8, lin2_W), jnp.tile(lin2_b, 8).reshape(1, 8))
    return s8.reshape(E)


# speedup vs baseline: 6.8597x; 1.0124x over previous
import numpy as np
import jax
import jax.numpy as jnp
from jax import lax
from jax.experimental import pallas as pl
from jax.experimental.pallas import tpu as pltpu
from jax.experimental.pallas import tpu_sc as plsc

N = 10000
NP = 10240
E = 160000
D = 128
H = 16
NC = 2
NS = 16
NW = NC * NS
EW = E // NW
CW = 40
CH = EW // CW
ROWS_N = NP // NS
TE = 6400
NT = 10
NB = N // NT

_mesh = plsc.VectorSubcoreMesh(core_axis_name="c", subcore_axis_name="s",
                               num_cores=NC, num_subcores=NS)


def _make_gather(n_pass, chp):
    ch = n_pass * chp
    rows_w = ch * CW
    rows_p = chp * CW

    def body(table_hbm, idx_hbm, out_hbm, idx_v, rows_v, sem):
        wid = lax.axis_index("s") * NC + lax.axis_index("c")
        pltpu.sync_copy(idx_hbm.at[wid], idx_v)
        for p in range(n_pass):
            def fire(j, carry):
                pltpu.async_copy(table_hbm.at[idx_v.at[p * chp + j]],
                                 rows_v.at[pl.ds(j * CW, CW)], sem)
                return carry
            lax.fori_loop(0, chp, fire, 0)
            base = wid * rows_w + p * rows_p
            pltpu.make_async_copy(out_hbm.at[pl.ds(base, rows_p)],
                                  rows_v, sem).wait()
            pltpu.sync_copy(rows_v, out_hbm.at[pl.ds(base, rows_p)])

    def run(table, idx3):
        return pl.kernel(
            body,
            out_type=jax.ShapeDtypeStruct((NW * rows_w, H), jnp.float32),
            mesh=_mesh,
            scratch_types=[pltpu.VMEM((ch, CW), jnp.int32),
                           pltpu.VMEM((rows_p, H), jnp.float32),
                           pltpu.SemaphoreType.DMA],
            compiler_params=pltpu.CompilerParams(use_tc_tiling_on_sc=False),
        )(table, idx3)
    return run


_gather_2e = _make_gather(2, CH)

EAW = EW // 8
EAC = EAW // 5


def _gather_xj_ea(table, src3, ea_pk):
    def body(table_hbm, idx_hbm, ea_hbm, out_hbm, ea_out, idx_v, rows_v,
             ea_v, sem):
        wid = lax.axis_index("s") * NC + lax.axis_index("c")
        pltpu.sync_copy(idx_hbm.at[wid], idx_v)

        def fire(j, carry):
            pltpu.async_copy(table_hbm.at[idx_v.at[j]],
                             rows_v.at[pl.ds(j * CW, CW)], sem)
            return carry
        lax.fori_loop(0, CH, fire, 0)

        def launder(t, carry):
            pltpu.sync_copy(ea_hbm.at[pl.ds(wid * EAW + t * EAC, EAC)], ea_v)
            pltpu.sync_copy(ea_v, ea_out.at[pl.ds(wid * EAW + t * EAC, EAC)])
            return carry
        lax.fori_loop(0, 5, launder, 0)
        base = wid * EW
        pltpu.make_async_copy(out_hbm.at[pl.ds(base, EW)], rows_v, sem).wait()
        pltpu.sync_copy(rows_v, out_hbm.at[pl.ds(base, EW)])

    return pl.kernel(
        body,
        out_type=[jax.ShapeDtypeStruct((E, H), jnp.float32),
                  jax.ShapeDtypeStruct((E // 8, 128), jnp.float32)],
        mesh=_mesh,
        scratch_types=[pltpu.VMEM((CH, CW), jnp.int32),
                       pltpu.VMEM((EW, H), jnp.float32),
                       pltpu.VMEM((EAC, 128), jnp.float32),
                       pltpu.SemaphoreType.DMA],
        compiler_params=pltpu.CompilerParams(use_tc_tiling_on_sc=False),
    )(table, src3, ea_pk)


def _scatter_add(msg, dst3, hr, zeros_nh):
    def body(msg_hbm, dst_hbm, hr_hbm, z_hbm, out_hbm, idx_v, msg_v, agg_sh):
        c = lax.axis_index("c")
        s = lax.axis_index("s")
        wid = s * NC + c
        @pl.when(jnp.logical_and(c == 0, s < NS - 1))
        def _():
            pltpu.sync_copy(hr_hbm.at[pl.ds(s * ROWS_N, ROWS_N)],
                            agg_sh.at[pl.ds(s * ROWS_N, ROWS_N)])

        @pl.when(jnp.logical_and(c == 0, s == NS - 1))
        def _():
            pltpu.sync_copy(hr_hbm.at[pl.ds((NS - 1) * ROWS_N,
                                            N - (NS - 1) * ROWS_N)],
                            agg_sh.at[pl.ds((NS - 1) * ROWS_N,
                                            N - (NS - 1) * ROWS_N)])
            pltpu.sync_copy(z_hbm.at[pl.ds(N, NP - N)],
                            agg_sh.at[pl.ds(N, NP - N)])

        @pl.when(c == 1)
        def _():
            pltpu.sync_copy(z_hbm.at[pl.ds(s * ROWS_N, ROWS_N)],
                            agg_sh.at[pl.ds(s * ROWS_N, ROWS_N)])
        pltpu.sync_copy(dst_hbm.at[wid], idx_v)
        pltpu.sync_copy(msg_hbm.at[pl.ds(wid * EW, EW)], msg_v)
        plsc.subcore_barrier()

        def sc(j, carry):
            pltpu.sync_copy(msg_v.at[pl.ds(j * CW, CW)],
                            agg_sh.at[idx_v.at[j]], add=True)
            return carry
        lax.fori_loop(0, CH, sc, 0)
        plsc.subcore_barrier()
        pltpu.sync_copy(agg_sh.at[pl.ds(s * ROWS_N, ROWS_N)],
                        out_hbm.at[c, pl.ds(s * ROWS_N, ROWS_N)])

    return pl.kernel(
        body,
        out_type=jax.ShapeDtypeStruct((NC, NP, H), jnp.float32),
        mesh=_mesh,
        scratch_types=[pltpu.VMEM((CH, CW), jnp.int32),
                       pltpu.VMEM((EW, H), jnp.float32),
                       pltpu.VMEM_SHARED((NP, H), jnp.float32)],
        compiler_params=pltpu.CompilerParams(use_tc_tiling_on_sc=False),
    )(msg, dst3, hr, zeros_nh)


def _tc_nodes(x, w0, b0, wr, br):
    def body(x_ref, w0_ref, b0_ref, wr_ref, br_ref, h_ref, hr_ref):
        h = jnp.maximum(
            jnp.dot(x_ref[...], w0_ref[...],
                    preferred_element_type=jnp.float32) + b0_ref[...], 0.0)
        h_ref[...] = h
        hr_ref[...] = jnp.dot(h, wr_ref[...],
                              preferred_element_type=jnp.float32) + br_ref[...]
    return pl.pallas_call(
        body,
        grid=(NT,),
        in_specs=[pl.BlockSpec((NB, D), lambda i: (i, 0)),
                  pl.BlockSpec((D, H), lambda i: (0, 0)),
                  pl.BlockSpec((1, H), lambda i: (0, 0)),
                  pl.BlockSpec((H, H), lambda i: (0, 0)),
                  pl.BlockSpec((1, H), lambda i: (0, 0))],
        out_specs=[pl.BlockSpec((NB, H), lambda i: (i, 0)),
                   pl.BlockSpec((NB, H), lambda i: (i, 0))],
        out_shape=[jax.ShapeDtypeStruct((N, H), jnp.float32),
                   jax.ShapeDtypeStruct((N, H), jnp.float32)],
    )(x, w0, b0, wr, br)


TEP = TE // 8


def _tc_msg(ea_pk, xj_pk, w1bd, b1t, w2s, b2bd):
    rkb = jnp.asarray(np.kron(
        np.eye(8, dtype=np.float32),
        np.repeat(np.eye(H, dtype=np.float32), H, axis=1)))
    rib = jnp.asarray(np.kron(
        np.eye(8, dtype=np.float32),
        np.tile(np.eye(H, dtype=np.float32), (1, H))))

    def body(ea_ref, xj_ref, w1_ref, b1_ref, w2s_ref, b2_ref, rkb_ref,
             rib_ref, msg_ref):
        zpk = jnp.maximum(
            jnp.dot(ea_ref[...], w1_ref[...],
                    preferred_element_type=jnp.float32) + b1_ref[...], 0.0)
        xjpk = xj_ref[...]
        p = (jnp.dot(zpk, rkb_ref[...], preferred_element_type=jnp.float32)
             * jnp.dot(xjpk, rib_ref[...], preferred_element_type=jnp.float32))
        msg_ref[...] = (
            jnp.dot(p, w2s_ref[...], preferred_element_type=jnp.float32)
            + jnp.dot(xjpk, b2_ref[...], preferred_element_type=jnp.float32))
    return pl.pallas_call(
        body,
        grid=(E // TE,),
        in_specs=[pl.BlockSpec((TEP, 128), lambda i: (i, 0)),
                  pl.BlockSpec((TEP, 128), lambda i: (i, 0)),
                  pl.BlockSpec((128, 128), lambda i: (0, 0)),
                  pl.BlockSpec((1, 128), lambda i: (0, 0)),
                  pl.BlockSpec((8 * H * H, 128), lambda i: (0, 0)),
                  pl.BlockSpec((128, 128), lambda i: (0, 0)),
                  pl.BlockSpec((128, 8 * H * H), lambda i: (0, 0)),
                  pl.BlockSpec((128, 8 * H * H), lambda i: (0, 0))],
        out_specs=pl.BlockSpec((TEP, 128), lambda i: (i, 0)),
        out_shape=jax.ShapeDtypeStruct((E // 8, 128), jnp.float32),
    )(ea_pk, xj_pk, w1bd, b1t, w2s, b2bd, rkb, rib)


def _tc_combine(agg2_pk):
    def body(a_ref, o_ref):
        o_ref[...] = a_ref[0] + a_ref[1]
    return pl.pallas_call(
        body,
        out_shape=jax.ShapeDtypeStruct((NP // 8, 128), jnp.float32),
    )(agg2_pk)


def _tc_head(r2_pk, w1bd, b1t, w2sel, b2):
    def body(xs_ref, xd_ref, w1_ref, b1_ref, w2_ref, b2_ref, o_ref):
        emb = xs_ref[...] * xd_ref[...]
        e1 = jnp.maximum(
            jnp.dot(emb, w1_ref[...],
                    preferred_element_type=jnp.float32) + b1_ref[...], 0.0)
        o_ref[...] = jnp.dot(e1, w2_ref[...],
                             preferred_element_type=jnp.float32) + b2_ref[...]
    nblk = E // TE
    return pl.pallas_call(
        body,
        grid=(nblk,),
        in_specs=[pl.BlockSpec((TEP, 128), lambda i: (i, 0)),
                  pl.BlockSpec((TEP, 128), lambda i: (i + nblk, 0)),
                  pl.BlockSpec((128, 64), lambda i: (0, 0)),
                  pl.BlockSpec((1, 64), lambda i: (0, 0)),
                  pl.BlockSpec((64, 8), lambda i: (0, 0)),
                  pl.BlockSpec((1, 8), lambda i: (0, 0))],
        out_specs=pl.BlockSpec((TEP, 8), lambda i: (i, 0)),
        out_shape=jax.ShapeDtypeStruct((E // 8, 8), jnp.float32),
    )(r2_pk, r2_pk, w1bd, b1t, w2sel, b2)


def kernel(x, edge_attr, lin0_W, lin0_b, enn_W1, enn_b1, enn_W2, enn_b2,
           root_W, conv_b, lin1_W, lin1_b, lin2_W, lin2_b, edge_index):
    src3 = edge_index[0].reshape(NW, CH, CW)
    dst3 = edge_index[1].reshape(NW, CH, CW)
    ei3 = edge_index.reshape(NW, 2 * CH, CW)
    i8 = jnp.eye(8, dtype=jnp.float32)

    h, hr = _tc_nodes(x, lin0_W, lin0_b.reshape(1, H),
                      root_W, conv_b.reshape(1, H))
    xj, ea_pk = _gather_xj_ea(h, src3, edge_attr.reshape(E // 8, 128))
    w2pad = jnp.pad(enn_W2.reshape(H * H, H), ((0, 0), (0, 128 - H)))
    w2s = jnp.concatenate([jnp.roll(w2pad, H * m, axis=1) for m in range(8)],
                          axis=0)
    msg_pk = _tc_msg(ea_pk, xj.reshape(E // 8, 128),
                     jnp.kron(i8, enn_W1), jnp.tile(enn_b1, 8).reshape(1, 128),
                     w2s, jnp.kron(i8, enn_b2.reshape(H, H)))
    agg2 = _scatter_add(msg_pk.reshape(E, H), dst3, hr,
                        jnp.zeros((NP, H), jnp.float32))
    out_pk = _tc_combine(agg2.reshape(NC, NP // 8, 128))
    r2 = _gather_2e(out_pk.reshape(NP, H), ei3)
    s8 = _tc_head(r2.reshape(2 * E // 8, 128),
                  jnp.kron(i8, lin1_W), jnp.tile(lin1_b, 8).reshape(1, 64),
                  jnp.kron(i8, lin2_W), jnp.tile(lin2_b, 8).reshape(1, 8))
    return s8.reshape(E)


# speedup vs baseline: 6.9140x; 1.0079x over previous
import numpy as np
import jax
import jax.numpy as jnp
from jax import lax
from jax.experimental import pallas as pl
from jax.experimental.pallas import tpu as pltpu
from jax.experimental.pallas import tpu_sc as plsc

N = 10000
NP = 10240
E = 160000
D = 128
H = 16
NC = 2
NS = 16
NW = NC * NS
EW = E // NW
CW = 40
CH = EW // CW
ROWS_N = NP // NS
TE = 6400
NT = 10
NB = N // NT

_mesh = plsc.VectorSubcoreMesh(core_axis_name="c", subcore_axis_name="s",
                               num_cores=NC, num_subcores=NS)


def _make_gather(n_pass, chp):
    ch = n_pass * chp
    rows_w = ch * CW
    rows_p = chp * CW

    def body(table_hbm, idx_hbm, out_hbm, idx_v, rows_v, sem):
        wid = lax.axis_index("s") * NC + lax.axis_index("c")
        pltpu.sync_copy(idx_hbm.at[wid], idx_v)
        for p in range(n_pass):
            def fire(j, carry):
                pltpu.async_copy(table_hbm.at[idx_v.at[p * chp + j]],
                                 rows_v.at[pl.ds(j * CW, CW)], sem)
                return carry
            lax.fori_loop(0, chp, fire, 0)
            base = wid * rows_w + p * rows_p
            pltpu.make_async_copy(out_hbm.at[pl.ds(base, rows_p)],
                                  rows_v, sem).wait()
            pltpu.sync_copy(rows_v, out_hbm.at[pl.ds(base, rows_p)])

    def run(table, idx3):
        return pl.kernel(
            body,
            out_type=jax.ShapeDtypeStruct((NW * rows_w, H), jnp.float32),
            mesh=_mesh,
            scratch_types=[pltpu.VMEM((ch, CW), jnp.int32),
                           pltpu.VMEM((rows_p, H), jnp.float32),
                           pltpu.SemaphoreType.DMA],
            compiler_params=pltpu.CompilerParams(use_tc_tiling_on_sc=False),
        )(table, idx3)
    return run


_gather_2e = _make_gather(2, CH)

EAC = EW // 5


def _gather_xj_ea(table, src3, ea):
    def body(table_hbm, idx_hbm, ea_hbm, out_hbm, ea_out, idx_v, rows_v,
             ea_v, sem):
        wid = lax.axis_index("s") * NC + lax.axis_index("c")
        pltpu.sync_copy(idx_hbm.at[wid], idx_v)

        def fire(j, carry):
            pltpu.async_copy(table_hbm.at[idx_v.at[j]],
                             rows_v.at[pl.ds(j * CW, CW)], sem)
            return carry
        lax.fori_loop(0, CH, fire, 0)

        def launder(t, carry):
            pltpu.sync_copy(ea_hbm.at[pl.ds(wid * EW + t * EAC, EAC)], ea_v)
            pltpu.sync_copy(ea_v, ea_out.at[pl.ds(wid * EW + t * EAC, EAC)])
            return carry
        lax.fori_loop(0, 5, launder, 0)
        base = wid * EW
        pltpu.make_async_copy(out_hbm.at[pl.ds(base, EW)], rows_v, sem).wait()
        pltpu.sync_copy(rows_v, out_hbm.at[pl.ds(base, EW)])

    return pl.kernel(
        body,
        out_type=[jax.ShapeDtypeStruct((E, H), jnp.float32),
                  jax.ShapeDtypeStruct((E, H), jnp.float32)],
        mesh=_mesh,
        scratch_types=[pltpu.VMEM((CH, CW), jnp.int32),
                       pltpu.VMEM((EW, H), jnp.float32),
                       pltpu.VMEM((EAC, H), jnp.float32),
                       pltpu.SemaphoreType.DMA],
        compiler_params=pltpu.CompilerParams(use_tc_tiling_on_sc=False),
    )(table, src3, ea)


def _scatter_add(msg, dst3, hr, zeros_nh):
    def body(msg_hbm, dst_hbm, hr_hbm, z_hbm, out_hbm, idx_v, msg_v, agg_sh):
        c = lax.axis_index("c")
        s = lax.axis_index("s")
        wid = s * NC + c
        @pl.when(jnp.logical_and(c == 0, s < NS - 1))
        def _():
            pltpu.sync_copy(hr_hbm.at[pl.ds(s * ROWS_N, ROWS_N)],
                            agg_sh.at[pl.ds(s * ROWS_N, ROWS_N)])

        @pl.when(jnp.logical_and(c == 0, s == NS - 1))
        def _():
            pltpu.sync_copy(hr_hbm.at[pl.ds((NS - 1) * ROWS_N,
                                            N - (NS - 1) * ROWS_N)],
                            agg_sh.at[pl.ds((NS - 1) * ROWS_N,
                                            N - (NS - 1) * ROWS_N)])
            pltpu.sync_copy(z_hbm.at[pl.ds(N, NP - N)],
                            agg_sh.at[pl.ds(N, NP - N)])

        @pl.when(c == 1)
        def _():
            pltpu.sync_copy(z_hbm.at[pl.ds(s * ROWS_N, ROWS_N)],
                            agg_sh.at[pl.ds(s * ROWS_N, ROWS_N)])
        pltpu.sync_copy(dst_hbm.at[wid], idx_v)
        pltpu.sync_copy(msg_hbm.at[pl.ds(wid * EW, EW)], msg_v)
        plsc.subcore_barrier()

        def sc(j, carry):
            pltpu.sync_copy(msg_v.at[pl.ds(j * CW, CW)],
                            agg_sh.at[idx_v.at[j]], add=True)
            return carry
        lax.fori_loop(0, CH, sc, 0)
        plsc.subcore_barrier()
        pltpu.sync_copy(agg_sh.at[pl.ds(s * ROWS_N, ROWS_N)],
                        out_hbm.at[c, pl.ds(s * ROWS_N, ROWS_N)])

    return pl.kernel(
        body,
        out_type=jax.ShapeDtypeStruct((NC, NP, H), jnp.float32),
        mesh=_mesh,
        scratch_types=[pltpu.VMEM((CH, CW), jnp.int32),
                       pltpu.VMEM((EW, H), jnp.float32),
                       pltpu.VMEM_SHARED((NP, H), jnp.float32)],
        compiler_params=pltpu.CompilerParams(use_tc_tiling_on_sc=False),
    )(msg, dst3, hr, zeros_nh)


def _tc_nodes(x, w0, b0, wr, br):
    def body(x_ref, w0_ref, b0_ref, wr_ref, br_ref, h_ref, hr_ref):
        h = jnp.maximum(
            jnp.dot(x_ref[...], w0_ref[...],
                    preferred_element_type=jnp.float32) + b0_ref[...], 0.0)
        h_ref[...] = h
        hr_ref[...] = jnp.dot(h, wr_ref[...],
                              preferred_element_type=jnp.float32) + br_ref[...]
    return pl.pallas_call(
        body,
        grid=(NT,),
        in_specs=[pl.BlockSpec((NB, D), lambda i: (i, 0)),
                  pl.BlockSpec((D, H), lambda i: (0, 0)),
                  pl.BlockSpec((1, H), lambda i: (0, 0)),
                  pl.BlockSpec((H, H), lambda i: (0, 0)),
                  pl.BlockSpec((1, H), lambda i: (0, 0))],
        out_specs=[pl.BlockSpec((NB, H), lambda i: (i, 0)),
                   pl.BlockSpec((NB, H), lambda i: (i, 0))],
        out_shape=[jax.ShapeDtypeStruct((N, H), jnp.float32),
                   jax.ShapeDtypeStruct((N, H), jnp.float32)],
    )(x, w0, b0, wr, br)


TEP = TE // 8


def _tc_msg(ea_pk, xj_pk, w1x, b1x, w2s, b2bd):
    rib = jnp.asarray(np.kron(
        np.eye(8, dtype=np.float32),
        np.tile(np.eye(H, dtype=np.float32), (1, H))))

    def body(ea_ref, xj_ref, w1x_ref, b1x_ref, w2s_ref, b2_ref, rib_ref,
             msg_ref):
        xjpk = xj_ref[...]
        zb = jnp.maximum(
            jnp.dot(ea_ref[...], w1x_ref[...],
                    preferred_element_type=jnp.float32) + b1x_ref[...], 0.0)
        p = zb * jnp.dot(xjpk, rib_ref[...],
                         preferred_element_type=jnp.float32)
        msg_ref[...] = (
            jnp.dot(p, w2s_ref[...], preferred_element_type=jnp.float32)
            + jnp.dot(xjpk, b2_ref[...], preferred_element_type=jnp.float32))
    return pl.pallas_call(
        body,
        grid=(E // TE,),
        in_specs=[pl.BlockSpec((TEP, 128), lambda i: (i, 0)),
                  pl.BlockSpec((TEP, 128), lambda i: (i, 0)),
                  pl.BlockSpec((128, 8 * H * H), lambda i: (0, 0)),
                  pl.BlockSpec((1, 8 * H * H), lambda i: (0, 0)),
                  pl.BlockSpec((8 * H * H, 128), lambda i: (0, 0)),
                  pl.BlockSpec((128, 128), lambda i: (0, 0)),
                  pl.BlockSpec((128, 8 * H * H), lambda i: (0, 0))],
        out_specs=pl.BlockSpec((TEP, 128), lambda i: (i, 0)),
        out_shape=jax.ShapeDtypeStruct((E // 8, 128), jnp.float32),
    )(ea_pk, xj_pk, w1x, b1x, w2s, b2bd, rib)


def _tc_combine(agg2_pk):
    def body(a_ref, o_ref):
        o_ref[...] = a_ref[0] + a_ref[1]
    return pl.pallas_call(
        body,
        out_shape=jax.ShapeDtypeStruct((NP // 8, 128), jnp.float32),
    )(agg2_pk)


def _tc_head(r2_pk, w1bd, b1t, w2sel, b2):
    def body(xs_ref, xd_ref, w1_ref, b1_ref, w2_ref, b2_ref, o_ref):
        emb = xs_ref[...] * xd_ref[...]
        e1 = jnp.maximum(
            jnp.dot(emb, w1_ref[...],
                    preferred_element_type=jnp.float32) + b1_ref[...], 0.0)
        o_ref[...] = jnp.dot(e1, w2_ref[...],
                             preferred_element_type=jnp.float32) + b2_ref[...]
    nblk = E // TE
    return pl.pallas_call(
        body,
        grid=(nblk,),
        in_specs=[pl.BlockSpec((TEP, 128), lambda i: (i, 0)),
                  pl.BlockSpec((TEP, 128), lambda i: (i + nblk, 0)),
                  pl.BlockSpec((128, 64), lambda i: (0, 0)),
                  pl.BlockSpec((1, 64), lambda i: (0, 0)),
                  pl.BlockSpec((64, 8), lambda i: (0, 0)),
                  pl.BlockSpec((1, 8), lambda i: (0, 0))],
        out_specs=pl.BlockSpec((TEP, 8), lambda i: (i, 0)),
        out_shape=jax.ShapeDtypeStruct((E // 8, 8), jnp.float32),
    )(r2_pk, r2_pk, w1bd, b1t, w2sel, b2)


def kernel(x, edge_attr, lin0_W, lin0_b, enn_W1, enn_b1, enn_W2, enn_b2,
           root_W, conv_b, lin1_W, lin1_b, lin2_W, lin2_b, edge_index):
    src3 = edge_index[0].reshape(NW, CH, CW)
    dst3 = edge_index[1].reshape(NW, CH, CW)
    ei3 = edge_index.reshape(NW, 2 * CH, CW)
    i8 = jnp.eye(8, dtype=jnp.float32)

    h, hr = _tc_nodes(x, lin0_W, lin0_b.reshape(1, H),
                      root_W, conv_b.reshape(1, H))
    xj, ea_lin = _gather_xj_ea(h, src3, edge_attr)
    w2pad = jnp.pad(enn_W2.reshape(H * H, H), ((0, 0), (0, 128 - H)))
    w2s = jnp.concatenate([jnp.roll(w2pad, H * m, axis=1) for m in range(8)],
                          axis=0)
    msg_pk = _tc_msg(ea_lin.reshape(E // 8, 128), xj.reshape(E // 8, 128),
                     jnp.kron(i8, jnp.repeat(enn_W1, H, axis=1)),
                     jnp.tile(jnp.repeat(enn_b1, H), 8).reshape(1, 8 * H * H),
                     w2s, jnp.kron(i8, enn_b2.reshape(H, H)))
    agg2 = _scatter_add(msg_pk.reshape(E, H), dst3, hr,
                        jnp.zeros((NP, H), jnp.float32))
    out_pk = _tc_combine(agg2.reshape(NC, NP // 8, 128))
    r2 = _gather_2e(out_pk.reshape(NP, H), ei3)
    s8 = _tc_head(r2.reshape(2 * E // 8, 128),
                  jnp.kron(i8, lin1_W), jnp.tile(lin1_b, 8).reshape(1, 64),
                  jnp.kron(i8, lin2_W), jnp.tile(lin2_b, 8).reshape(1, 8))
    return s8.reshape(E)
